# Initial kernel scaffold; baseline (speedup 1.0000x reference)
import jax
import jax.numpy as jnp
from jax.experimental import pallas as pl


def kernel(x, edge_index, edge_attrs, W1_w, W1_b):
    raise NotImplementedError("write your pallas kernel here")


# speedup vs baseline: 16.6728x; 16.6728x over previous
import functools

import jax
import jax.numpy as jnp
from jax import lax
from jax.experimental import pallas as pl
from jax.experimental.pallas import tpu as pltpu
from jax.experimental.pallas import tpu_sc as plsc

NC = 2
NS = 16
NW = NC * NS
CH = 128


def _round_up(a, b):
    return (a + b - 1) // b * b


def _deg_kernel(n_pad, k):
    rpt = n_pad // NS
    mesh = plsc.VectorSubcoreMesh(core_axis_name="c", subcore_axis_name="s")

    @functools.partial(
        pl.kernel,
        out_type=jax.ShapeDtypeStruct((NC * n_pad,), jnp.float32),
        mesh=mesh,
        scratch_types=[
            pltpu.VMEM((k, CH), jnp.int32),
            pltpu.VMEM((CH,), jnp.float32),
            pltpu.VMEM((rpt,), jnp.float32),
            pltpu.VMEM_SHARED((n_pad,), jnp.float32),
        ],
    )
    def deg_kernel(to_hbm, out_hbm, idx_v, ones_v, zbuf_v, deg_sh):
        c = lax.axis_index("c")
        s = lax.axis_index("s")
        wid = c * NS + s
        pltpu.sync_copy(to_hbm.at[wid], idx_v)
        for i in range(CH // 16):
            ones_v[pl.ds(i * 16, 16)] = jnp.ones((16,), jnp.float32)

        def zb(i, carry):
            zbuf_v[pl.ds(i * 16, 16)] = jnp.zeros((16,), jnp.float32)
            return carry

        lax.fori_loop(0, rpt // 16, zb, 0)
        pltpu.sync_copy(zbuf_v, deg_sh.at[pl.ds(s * rpt, rpt)])
        plsc.subcore_barrier()

        def body(j, carry):
            pltpu.sync_copy(ones_v, deg_sh.at[idx_v.at[j]], add=True)
            return carry

        lax.fori_loop(0, k, body, 0)
        plsc.subcore_barrier()
        pltpu.sync_copy(deg_sh.at[pl.ds(s * rpt, rpt)],
                        out_hbm.at[pl.ds(c * n_pad + s * rpt, rpt)])

    return deg_kernel


def _gather_scatter_kernel(n_pad, k, d):
    rpt = n_pad // NS
    mesh = plsc.VectorSubcoreMesh(core_axis_name="c", subcore_axis_name="s")

    @functools.partial(
        pl.kernel,
        out_type=(
            jax.ShapeDtypeStruct((NC, n_pad, d), jnp.float32),
            jax.ShapeDtypeStruct((NC * n_pad,), jnp.float32),
        ),
        mesh=mesh,
        scratch_types=[
            pltpu.VMEM((k, CH), jnp.int32),
            pltpu.VMEM((k, CH), jnp.int32),
            pltpu.VMEM((CH, d), jnp.float32),
            pltpu.VMEM((CH,), jnp.float32),
            pltpu.VMEM_SHARED((n_pad, d), jnp.float32),
            pltpu.VMEM_SHARED((n_pad,), jnp.float32),
        ],
    )
    def gs_kernel(from_hbm, to_hbm, y_hbm, dinv_hbm, gout_hbm, sout_hbm,
                  fidx_v, tidx_v, rows_v, drow_v, g_sh, s_sh):
        c = lax.axis_index("c")
        s = lax.axis_index("s")
        wid = c * NS + s
        pltpu.sync_copy(from_hbm.at[wid], fidx_v)
        pltpu.sync_copy(to_hbm.at[wid], tidx_v)

        def zr(i, carry):
            def zc(j, inner):
                rows_v[i, pl.ds(j * 16, 16)] = jnp.zeros((16,), jnp.float32)
                return inner
            return lax.fori_loop(0, d // 16, zc, carry)

        lax.fori_loop(0, CH, zr, 0)
        for i in range(CH // 16):
            drow_v[pl.ds(i * 16, 16)] = jnp.zeros((16,), jnp.float32)
        for q in range(rpt // CH):
            pltpu.sync_copy(rows_v, g_sh.at[pl.ds(s * rpt + q * CH, CH)])
            pltpu.sync_copy(drow_v, s_sh.at[pl.ds(s * rpt + q * CH, CH)])
        plsc.subcore_barrier()

        def body(j, carry):
            pltpu.sync_copy(y_hbm.at[fidx_v.at[j]], rows_v)
            pltpu.sync_copy(rows_v, g_sh.at[tidx_v.at[j]], add=True)
            pltpu.sync_copy(dinv_hbm.at[fidx_v.at[j]], drow_v)
            pltpu.sync_copy(drow_v, s_sh.at[tidx_v.at[j]], add=True)
            return carry

        lax.fori_loop(0, k, body, 0)
        plsc.subcore_barrier()
        pltpu.sync_copy(g_sh.at[pl.ds(s * rpt, rpt)],
                        gout_hbm.at[c, pl.ds(s * rpt, rpt)])
        pltpu.sync_copy(s_sh.at[pl.ds(s * rpt, rpt)],
                        sout_hbm.at[pl.ds(c * n_pad + s * rpt, rpt)])

    return gs_kernel


def _build_y_kernel(n_pad, d, blk):
    def body(dp_ref, x_ref, y_ref, di_ref):
        deg = dp_ref[:, 0:1] + dp_ref[:, 1:2]
        deg_inv = jnp.where(deg > 0.0, lax.rsqrt(jnp.maximum(deg, 1.0e-12)), 0.0)
        y_ref[...] = x_ref[...] * deg_inv
        di_ref[...] = deg_inv

    grid = n_pad // blk
    return pl.pallas_call(
        body,
        grid=(grid,),
        in_specs=[
            pl.BlockSpec((blk, NC), lambda i: (i, 0)),
            pl.BlockSpec((blk, d), lambda i: (i, 0)),
        ],
        out_specs=[
            pl.BlockSpec((blk, d), lambda i: (i, 0)),
            pl.BlockSpec((blk, 1), lambda i: (i, 0)),
        ],
        out_shape=[
            jax.ShapeDtypeStruct((n_pad, d), jnp.float32),
            jax.ShapeDtypeStruct((n_pad, 1), jnp.float32),
        ],
    )


def _final_kernel(n_pad, d, blk):
    def body(dp_ref, sp_ref, x_ref, g_ref, wt_ref, b_ref, o_ref):
        deg = dp_ref[:, 0:1] + dp_ref[:, 1:2]
        deg_inv = jnp.where(deg > 0.0, lax.rsqrt(jnp.maximum(deg, 1.0e-12)), 0.0)
        g = g_ref[0] + g_ref[1]
        sv = sp_ref[:, 0:1] + sp_ref[:, 1:2]
        x = x_ref[...]
        t = x * g
        lin = jnp.dot(t, wt_ref[...], preferred_element_type=jnp.float32,
                      precision=lax.Precision.HIGHEST)
        u = sv * (x + b_ref[...]) + lin
        v = deg_inv * u
        o_ref[...] = jnp.where(v >= 0.0, v, 0.01 * v)

    grid = n_pad // blk
    return pl.pallas_call(
        body,
        grid=(grid,),
        in_specs=[
            pl.BlockSpec((blk, NC), lambda i: (i, 0)),
            pl.BlockSpec((blk, NC), lambda i: (i, 0)),
            pl.BlockSpec((blk, d), lambda i: (i, 0)),
            pl.BlockSpec((NC, blk, d), lambda i: (0, i, 0)),
            pl.BlockSpec((d, d), lambda i: (0, 0)),
            pl.BlockSpec((1, d), lambda i: (0, 0)),
        ],
        out_specs=pl.BlockSpec((blk, d), lambda i: (i, 0)),
        out_shape=jax.ShapeDtypeStruct((n_pad, d), jnp.float32),
    )


def kernel(x, edge_index, edge_attrs, W1_w, W1_b):
    n, d = x.shape
    e = edge_index.shape[1]

    k = _round_up(e, NW * CH) // (NW * CH)
    e_pad = NW * k * CH
    n_pad = _round_up(n + 1, NS * CH)

    from_ = edge_index[0].astype(jnp.int32)
    to_ = edge_index[1].astype(jnp.int32)
    padv = jnp.full((e_pad - e,), n, jnp.int32)
    from_t = jnp.concatenate([from_, padv]).reshape(NW, k, CH)
    to_t = jnp.concatenate([to_, padv]).reshape(NW, k, CH)
    x_pad = jnp.pad(x, ((0, n_pad - n), (0, 0)))

    deg_parts = _deg_kernel(n_pad, k)(to_t).reshape(NC, n_pad)
    dp_t = deg_parts.T

    blk = n_pad // 16
    y, dinv = _build_y_kernel(n_pad, d, blk)(dp_t, x_pad)
    dinv1 = dinv.reshape(n_pad)

    g_parts, s_parts = _gather_scatter_kernel(n_pad, k, d)(
        from_t, to_t, y, dinv1)
    sp_t = s_parts.reshape(NC, n_pad).T

    wt = W1_w.T
    b2 = W1_b.reshape(1, d)
    out = _final_kernel(n_pad, d, blk)(dp_t, sp_t, x_pad, g_parts, wt, b2)
    return out[:n]
